# baseline (speedup 1.0000x reference)
import functools

import jax
import jax.numpy as jnp
from jax import lax
from jax.experimental import pallas as pl
from jax.experimental.pallas import tpu as pltpu
from jax.experimental.pallas import tpu_sc as plsc

_N = 10000
_E = 320000
_H = 128
_G = 64

_NC = 2
_NS = 16
_NW = _NC * _NS
_EPW = _E // _NW
_CH = 80
_NCHUNK = _EPW // _CH
_NB = 3
_NP = 10240
_RPT = _NP // _NS

_mesh = plsc.VectorSubcoreMesh(
    core_axis_name="c", subcore_axis_name="s", num_cores=_NC, num_subcores=_NS
)


def _deg_body(eflat_hbm, out_hbm, dst_v, deg_v):
    c = lax.axis_index("c")
    s = lax.axis_index("s")
    wid = c * _NS + s
    base = _E + wid * _EPW
    pltpu.sync_copy(eflat_hbm.at[pl.ds(base, _EPW)], dst_v)

    zero = jnp.zeros((16,), jnp.float32)

    def zb(i, carry):
        deg_v[pl.ds(i * 16, 16)] = zero
        return carry

    lax.fori_loop(0, _N // 16, zb, 0)

    ones = jnp.ones((16,), jnp.float32)

    def body(i, carry):
        idx = dst_v[pl.ds(i * 16, 16)]
        plsc.addupdate_scatter(deg_v, [idx], ones)
        return carry

    lax.fori_loop(0, _EPW // 16, body, 0)
    pltpu.sync_copy(deg_v, out_hbm.at[wid])


_deg_call = functools.partial(
    pl.kernel,
    out_type=jax.ShapeDtypeStruct((_NW, _N), jnp.float32),
    mesh=_mesh,
    scratch_types=[
        pltpu.VMEM((_EPW,), jnp.int32),
        pltpu.VMEM((_N,), jnp.float32),
    ],
    compiler_params=pltpu.CompilerParams(needs_layout_passes=False),
)(_deg_body)


def _scat_body(hp_hbm, eflat_hbm, out_hbm, sidx_v, didx_v,
               rows0, rows1, rows2, acc_sh,
               gsem0, gsem1, gsem2, ssem0, ssem1, ssem2,
               isem0, isem1, isem2):
    c = lax.axis_index("c")
    s = lax.axis_index("s")
    wid = c * _NS + s
    base = wid * _EPW

    pltpu.sync_copy(eflat_hbm.at[pl.ds(base, _EPW)], sidx_v)

    rows = (rows0, rows1, rows2)
    gsem = (gsem0, gsem1, gsem2)
    ssem = (ssem0, ssem1, ssem2)
    isem = (isem0, isem1, isem2)

    def gather(k, b):
        pltpu.async_copy(hp_hbm.at[sidx_v.at[pl.ds(k * _CH, _CH)]],
                         rows[b], gsem[b])

    def dfetch(k, b):
        pltpu.async_copy(
            eflat_hbm.at[pl.ds(_E + base + k * _CH, _CH)],
            didx_v.at[b], isem[b])

    zero = jnp.zeros((16,), jnp.float32)

    def zb(i, carry):
        for j in range(8):
            rows0[i, pl.ds(j * 16, 16)] = zero
            rows1[i, pl.ds(j * 16, 16)] = zero
        return carry

    lax.fori_loop(0, _CH, zb, 0)
    row0 = s * _RPT
    for k in range(_RPT // (2 * _CH)):
        pltpu.sync_copy(rows0, acc_sh.at[pl.ds(row0 + 2 * k * _CH, _CH)])
        pltpu.sync_copy(rows1, acc_sh.at[pl.ds(row0 + (2 * k + 1) * _CH, _CH)])

    for b in range(_NB):
        gather(b, b)
        dfetch(b, b)
    plsc.subcore_barrier()

    def step(k, b):
        pb = (b - 1) % _NB
        pltpu.make_async_copy(hp_hbm.at[sidx_v.at[pl.ds(0, _CH)]],
                              rows[b], gsem[b]).wait()
        pltpu.make_async_copy(eflat_hbm.at[pl.ds(0, _CH)],
                              didx_v.at[b], isem[b]).wait()
        pltpu.async_copy(rows[b], acc_sh.at[didx_v.at[b]], ssem[b], add=True)

        @pl.when(k >= 1)
        def _():
            pltpu.make_async_copy(rows[pb], acc_sh.at[didx_v.at[pb]],
                                  ssem[pb]).wait()

            @pl.when(k - 1 + _NB < _NCHUNK)
            def _():
                gather(k - 1 + _NB, pb)
                dfetch(k - 1 + _NB, pb)

    def hexstep(j, carry):
        for b in range(_NB):
            step(_NB * j + b, b)
        return carry

    lax.fori_loop(0, _NCHUNK // _NB, hexstep, 0)
    for t in range(_NCHUNK - _NCHUNK % _NB, _NCHUNK):
        step(t, t % _NB)
    lb = (_NCHUNK - 1) % _NB
    pltpu.make_async_copy(rows[lb], acc_sh.at[didx_v.at[lb]], ssem[lb]).wait()
    plsc.subcore_barrier()

    pltpu.sync_copy(
        acc_sh.at[pl.ds(row0, _RPT)], out_hbm.at[c, pl.ds(row0, _RPT)]
    )


_scat_call = functools.partial(
    pl.kernel,
    out_type=jax.ShapeDtypeStruct((_NC, _NP, _H), jnp.float32),
    mesh=_mesh,
    scratch_types=[
        pltpu.VMEM((_EPW,), jnp.int32),
        pltpu.VMEM((_NB, _CH), jnp.int32),
        pltpu.VMEM((_CH, _H), jnp.float32),
        pltpu.VMEM((_CH, _H), jnp.float32),
        pltpu.VMEM((_CH, _H), jnp.float32),
        pltpu.VMEM_SHARED((_NP, _H), jnp.float32),
    ] + [pltpu.SemaphoreType.DMA] * 9,
)(_scat_body)


_BR = 1000
_GRID = _N // _BR

_PREC = jax.lax.Precision.HIGHEST


def _mm1_body(x_ref, w_ref, disb_ref, o_ref):
    h = jnp.dot(x_ref[...], w_ref[...], precision=_PREC,
                preferred_element_type=jnp.float32)
    o_ref[...] = disb_ref[...] * h


def _mm1_call(x, W1, disb):
    return pl.pallas_call(
        _mm1_body,
        grid=(_GRID,),
        in_specs=[
            pl.BlockSpec((_BR, _H), lambda i: (i, 0)),
            pl.BlockSpec((_H, _H), lambda i: (0, 0)),
            pl.BlockSpec((_BR, _H), lambda i: (i, 0)),
        ],
        out_specs=pl.BlockSpec((_BR, _H), lambda i: (i, 0)),
        out_shape=jax.ShapeDtypeStruct((_N, _H), jnp.float32),
    )(x, W1, disb)


def _mm2_body(p_ref, hp_ref, disb_ref, b_ref, w_ref, o_ref):
    disb = disb_ref[...]
    z = disb * (p_ref[0] + p_ref[1] + hp_ref[...]) + b_ref[...]
    z = jnp.maximum(z, 0.0)
    h = jnp.dot(z, w_ref[...], precision=_PREC,
                preferred_element_type=jnp.float32)
    o_ref[...] = disb * h


def _mm2_call(p, hp, disb, b, W):
    blk = pl.BlockSpec((_BR, _H), lambda i: (i, 0))
    return pl.pallas_call(
        _mm2_body,
        grid=(_GRID,),
        in_specs=[
            pl.BlockSpec((_NC, _BR, _H), lambda i: (0, i, 0)),
            blk, blk,
            pl.BlockSpec((1, _H), lambda i: (0, 0)),
            pl.BlockSpec((_H, _H), lambda i: (0, 0)),
        ],
        out_specs=blk,
        out_shape=jax.ShapeDtypeStruct((_N, _H), jnp.float32),
    )(p, hp, disb, b, W)


def _mm3_body(q_ref, hp_ref, disb_ref, b_ref, batch_ref, wp_ref,
              bp_ref, o_ref, acc, cnt):
    i = pl.program_id(0)

    @pl.when(i == 0)
    def _():
        acc[...] = jnp.zeros_like(acc)
        cnt[...] = jnp.zeros_like(cnt)

    z = disb_ref[...] * (q_ref[0] + q_ref[1] + hp_ref[...]) + b_ref[...]
    z = jnp.maximum(z, 0.0)

    bvec = batch_ref[0]
    gids = lax.broadcasted_iota(jnp.int32, (_G, 1), 0)
    onehot_t = (bvec == gids).astype(jnp.float32)
    acc[...] += jnp.dot(onehot_t, z, precision=_PREC,
                        preferred_element_type=jnp.float32)
    cnt[...] += jnp.dot(onehot_t, jnp.ones((_BR, _H), jnp.float32),
                        precision=_PREC, preferred_element_type=jnp.float32)

    @pl.when(i == _GRID - 1)
    def _():
        pooled = acc[...] / jnp.maximum(cnt[...], 1.0)
        o_ref[...] = jnp.dot(pooled, wp_ref[...], precision=_PREC,
                             preferred_element_type=jnp.float32) + bp_ref[...]


def _mm3_call(q, hp, disb, b, batch3, Wp, bp):
    blk = pl.BlockSpec((_BR, _H), lambda i: (i, 0))
    row = pl.BlockSpec((1, _H), lambda i: (0, 0))
    return pl.pallas_call(
        _mm3_body,
        grid=(_GRID,),
        in_specs=[
            pl.BlockSpec((_NC, _BR, _H), lambda i: (0, i, 0)),
            blk, blk, row,
            pl.BlockSpec((1, 1, _BR), lambda i: (i, 0, 0)),
            pl.BlockSpec((_H, _H), lambda i: (0, 0)),
            row,
        ],
        out_specs=pl.BlockSpec((_G, _H), lambda i: (0, 0)),
        out_shape=jax.ShapeDtypeStruct((_G, _H), jnp.float32),
        scratch_shapes=[
            pltpu.VMEM((_G, _H), jnp.float32),
            pltpu.VMEM((_G, _H), jnp.float32),
        ],
    )(q, hp, disb, b, batch3, Wp, bp)


def kernel(x, edge_index, batch, W1, b1, W2, b2, Wp, bp):
    eflat = edge_index.reshape(2 * _E)
    degp = _deg_call(eflat)
    deg = jnp.sum(degp, axis=0) + 1.0
    disb = jnp.broadcast_to(lax.rsqrt(deg)[:, None], (_N, _H))

    h1p = _mm1_call(x, W1, disb)
    p = _scat_call(h1p, eflat)
    h2p = _mm2_call(p, h1p, disb, b1.reshape(1, _H), W2)
    q = _scat_call(h2p, eflat)
    out = _mm3_call(q, h2p, disb, b2.reshape(1, _H),
                    batch.reshape(_GRID, 1, _BR), Wp, bp.reshape(1, _H))
    return out


# speedup vs baseline: 1.0520x; 1.0520x over previous
import functools

import jax
import jax.numpy as jnp
from jax import lax
from jax.experimental import pallas as pl
from jax.experimental.pallas import tpu as pltpu
from jax.experimental.pallas import tpu_sc as plsc

_N = 10000
_E = 320000
_H = 128
_G = 64

_NC = 2
_NS = 16
_NW = _NC * _NS
_EPW = _E // _NW
_CH = 40
_NCHUNK = _EPW // _CH
_NB = 7
_NP = 10240
_RPT = _NP // _NS

_mesh = plsc.VectorSubcoreMesh(
    core_axis_name="c", subcore_axis_name="s", num_cores=_NC, num_subcores=_NS
)


def _deg_body(eflat_hbm, out_hbm, dst_v, deg_v):
    c = lax.axis_index("c")
    s = lax.axis_index("s")
    wid = c * _NS + s
    base = _E + wid * _EPW
    pltpu.sync_copy(eflat_hbm.at[pl.ds(base, _EPW)], dst_v)

    zero = jnp.zeros((16,), jnp.float32)

    def zb(i, carry):
        deg_v[pl.ds(i * 16, 16)] = zero
        return carry

    lax.fori_loop(0, _N // 16, zb, 0)

    ones = jnp.ones((16,), jnp.float32)

    def body(i, carry):
        idx = dst_v[pl.ds(i * 16, 16)]
        plsc.addupdate_scatter(deg_v, [idx], ones)
        return carry

    lax.fori_loop(0, _EPW // 16, body, 0)
    pltpu.sync_copy(deg_v, out_hbm.at[wid])


_deg_call = functools.partial(
    pl.kernel,
    out_type=jax.ShapeDtypeStruct((_NW, _N), jnp.float32),
    mesh=_mesh,
    scratch_types=[
        pltpu.VMEM((_EPW,), jnp.int32),
        pltpu.VMEM((_N,), jnp.float32),
    ],
    compiler_params=pltpu.CompilerParams(needs_layout_passes=False),
)(_deg_body)


def _scat_body(hp_hbm, eflat_hbm, out_hbm, sidx_v, didx_v,
               rows0, rows1, rows2, rows3, rows4, rows5, rows6, acc_sh,
               gsem0, gsem1, gsem2, gsem3, gsem4, gsem5, gsem6,
               ssem0, ssem1, ssem2, ssem3, ssem4, ssem5, ssem6,
               isem0, isem1, isem2, isem3, isem4, isem5, isem6):
    c = lax.axis_index("c")
    s = lax.axis_index("s")
    wid = c * _NS + s
    base = wid * _EPW

    pltpu.sync_copy(eflat_hbm.at[pl.ds(base, _EPW)], sidx_v)

    rows = (rows0, rows1, rows2, rows3, rows4, rows5, rows6)
    gsem = (gsem0, gsem1, gsem2, gsem3, gsem4, gsem5, gsem6)
    ssem = (ssem0, ssem1, ssem2, ssem3, ssem4, ssem5, ssem6)
    isem = (isem0, isem1, isem2, isem3, isem4, isem5, isem6)

    def gather(k, b):
        pltpu.async_copy(hp_hbm.at[sidx_v.at[pl.ds(k * _CH, _CH)]],
                         rows[b], gsem[b])

    def dfetch(k, b):
        pltpu.async_copy(
            eflat_hbm.at[pl.ds(_E + base + k * _CH, _CH)],
            didx_v.at[b], isem[b])

    zero = jnp.zeros((16,), jnp.float32)

    def zb(i, carry):
        for j in range(8):
            rows0[i, pl.ds(j * 16, 16)] = zero
            rows1[i, pl.ds(j * 16, 16)] = zero
        return carry

    lax.fori_loop(0, _CH, zb, 0)
    row0 = s * _RPT
    for k in range(_RPT // (2 * _CH)):
        pltpu.sync_copy(rows0, acc_sh.at[pl.ds(row0 + 2 * k * _CH, _CH)])
        pltpu.sync_copy(rows1, acc_sh.at[pl.ds(row0 + (2 * k + 1) * _CH, _CH)])

    for b in range(_NB):
        gather(b, b)
        dfetch(b, b)
    plsc.subcore_barrier()

    def step(k, b):
        pb = (b - 1) % _NB
        pltpu.make_async_copy(hp_hbm.at[sidx_v.at[pl.ds(0, _CH)]],
                              rows[b], gsem[b]).wait()
        pltpu.make_async_copy(eflat_hbm.at[pl.ds(0, _CH)],
                              didx_v.at[b], isem[b]).wait()
        pltpu.async_copy(rows[b], acc_sh.at[didx_v.at[b]], ssem[b], add=True)

        @pl.when(k >= 1)
        def _():
            pltpu.make_async_copy(rows[pb], acc_sh.at[didx_v.at[pb]],
                                  ssem[pb]).wait()

            @pl.when(k - 1 + _NB < _NCHUNK)
            def _():
                gather(k - 1 + _NB, pb)
                dfetch(k - 1 + _NB, pb)

    def hexstep(j, carry):
        for b in range(_NB):
            step(_NB * j + b, b)
        return carry

    lax.fori_loop(0, _NCHUNK // _NB, hexstep, 0)
    for t in range(_NCHUNK - _NCHUNK % _NB, _NCHUNK):
        step(t, t % _NB)
    lb = (_NCHUNK - 1) % _NB
    pltpu.make_async_copy(rows[lb], acc_sh.at[didx_v.at[lb]], ssem[lb]).wait()
    plsc.subcore_barrier()

    pltpu.sync_copy(
        acc_sh.at[pl.ds(row0, _RPT)], out_hbm.at[c, pl.ds(row0, _RPT)]
    )


_scat_call = functools.partial(
    pl.kernel,
    out_type=jax.ShapeDtypeStruct((_NC, _NP, _H), jnp.float32),
    mesh=_mesh,
    scratch_types=[
        pltpu.VMEM((_EPW,), jnp.int32),
        pltpu.VMEM((_NB, _CH), jnp.int32),
        pltpu.VMEM((_CH, _H), jnp.float32),
        pltpu.VMEM((_CH, _H), jnp.float32),
        pltpu.VMEM((_CH, _H), jnp.float32),
        pltpu.VMEM((_CH, _H), jnp.float32),
        pltpu.VMEM((_CH, _H), jnp.float32),
        pltpu.VMEM((_CH, _H), jnp.float32),
        pltpu.VMEM((_CH, _H), jnp.float32),
        pltpu.VMEM_SHARED((_NP, _H), jnp.float32),
    ] + [pltpu.SemaphoreType.DMA] * 21,
)(_scat_body)


_BR = 1000
_GRID = _N // _BR

_PREC = jax.lax.Precision.HIGHEST


def _mm1_body(x_ref, w_ref, disb_ref, o_ref):
    h = jnp.dot(x_ref[...], w_ref[...], precision=_PREC,
                preferred_element_type=jnp.float32)
    o_ref[...] = disb_ref[...] * h


def _mm1_call(x, W1, disb):
    return pl.pallas_call(
        _mm1_body,
        grid=(_GRID,),
        in_specs=[
            pl.BlockSpec((_BR, _H), lambda i: (i, 0)),
            pl.BlockSpec((_H, _H), lambda i: (0, 0)),
            pl.BlockSpec((_BR, _H), lambda i: (i, 0)),
        ],
        out_specs=pl.BlockSpec((_BR, _H), lambda i: (i, 0)),
        out_shape=jax.ShapeDtypeStruct((_N, _H), jnp.float32),
    )(x, W1, disb)


def _mm2_body(p_ref, hp_ref, disb_ref, b_ref, w_ref, o_ref):
    disb = disb_ref[...]
    z = disb * (p_ref[0] + p_ref[1] + hp_ref[...]) + b_ref[...]
    z = jnp.maximum(z, 0.0)
    h = jnp.dot(z, w_ref[...], precision=_PREC,
                preferred_element_type=jnp.float32)
    o_ref[...] = disb * h


def _mm2_call(p, hp, disb, b, W):
    blk = pl.BlockSpec((_BR, _H), lambda i: (i, 0))
    return pl.pallas_call(
        _mm2_body,
        grid=(_GRID,),
        in_specs=[
            pl.BlockSpec((_NC, _BR, _H), lambda i: (0, i, 0)),
            blk, blk,
            pl.BlockSpec((1, _H), lambda i: (0, 0)),
            pl.BlockSpec((_H, _H), lambda i: (0, 0)),
        ],
        out_specs=blk,
        out_shape=jax.ShapeDtypeStruct((_N, _H), jnp.float32),
    )(p, hp, disb, b, W)


def _mm3_body(q_ref, hp_ref, disb_ref, b_ref, batch_ref, wp_ref,
              bp_ref, o_ref, acc, cnt):
    i = pl.program_id(0)

    @pl.when(i == 0)
    def _():
        acc[...] = jnp.zeros_like(acc)
        cnt[...] = jnp.zeros_like(cnt)

    z = disb_ref[...] * (q_ref[0] + q_ref[1] + hp_ref[...]) + b_ref[...]
    z = jnp.maximum(z, 0.0)

    bvec = batch_ref[0]
    gids = lax.broadcasted_iota(jnp.int32, (_G, 1), 0)
    onehot_t = (bvec == gids).astype(jnp.float32)
    acc[...] += jnp.dot(onehot_t, z, precision=_PREC,
                        preferred_element_type=jnp.float32)
    cnt[...] += jnp.dot(onehot_t, jnp.ones((_BR, _H), jnp.float32),
                        precision=_PREC, preferred_element_type=jnp.float32)

    @pl.when(i == _GRID - 1)
    def _():
        pooled = acc[...] / jnp.maximum(cnt[...], 1.0)
        o_ref[...] = jnp.dot(pooled, wp_ref[...], precision=_PREC,
                             preferred_element_type=jnp.float32) + bp_ref[...]


def _mm3_call(q, hp, disb, b, batch3, Wp, bp):
    blk = pl.BlockSpec((_BR, _H), lambda i: (i, 0))
    row = pl.BlockSpec((1, _H), lambda i: (0, 0))
    return pl.pallas_call(
        _mm3_body,
        grid=(_GRID,),
        in_specs=[
            pl.BlockSpec((_NC, _BR, _H), lambda i: (0, i, 0)),
            blk, blk, row,
            pl.BlockSpec((1, 1, _BR), lambda i: (i, 0, 0)),
            pl.BlockSpec((_H, _H), lambda i: (0, 0)),
            row,
        ],
        out_specs=pl.BlockSpec((_G, _H), lambda i: (0, 0)),
        out_shape=jax.ShapeDtypeStruct((_G, _H), jnp.float32),
        scratch_shapes=[
            pltpu.VMEM((_G, _H), jnp.float32),
            pltpu.VMEM((_G, _H), jnp.float32),
        ],
    )(q, hp, disb, b, batch3, Wp, bp)


def kernel(x, edge_index, batch, W1, b1, W2, b2, Wp, bp):
    eflat = edge_index.reshape(2 * _E)
    degp = _deg_call(eflat)
    deg = jnp.sum(degp, axis=0) + 1.0
    disb = jnp.broadcast_to(lax.rsqrt(deg)[:, None], (_N, _H))

    h1p = _mm1_call(x, W1, disb)
    p = _scat_call(h1p, eflat)
    h2p = _mm2_call(p, h1p, disb, b1.reshape(1, _H), W2)
    q = _scat_call(h2p, eflat)
    out = _mm3_call(q, h2p, disb, b2.reshape(1, _H),
                    batch.reshape(_GRID, 1, _BR), Wp, bp.reshape(1, _H))
    return out
